# baseline (speedup 1.0000x reference)
import functools

import jax
import jax.numpy as jnp
from jax import lax
from jax.experimental import pallas as pl
from jax.experimental.pallas import tpu as pltpu
from jax.experimental.pallas import tpu_sc as plsc

_NC = 2
_NS = 16
_NW = _NC * _NS
_CHUNK = 256
_ZSTG = 128


def _round_up(a, b):
    return (a + b - 1) // b * b


@functools.lru_cache(maxsize=None)
def _seg_sum_kernel(n_nodes, n_chunks, emb):
    rows_sp = _round_up(n_nodes + 1, _NS * _ZSTG)
    n_out = rows_sp // _NS
    nz = n_out // _ZSTG

    @functools.partial(
        pl.kernel,
        out_type=jax.ShapeDtypeStruct((_NC, rows_sp, emb), jnp.float32),
        mesh=plsc.VectorSubcoreMesh(core_axis_name="c", subcore_axis_name="s"),
        scratch_types=[
            pltpu.VMEM((n_chunks, _CHUNK), jnp.int32),
            pltpu.VMEM((_CHUNK,), jnp.int32),
            pltpu.VMEM((_CHUNK,), jnp.int32),
            pltpu.VMEM((_CHUNK, emb), jnp.float32),
            pltpu.VMEM_SHARED((rows_sp, emb), jnp.float32),
            pltpu.SemaphoreType.DMA,
            pltpu.SemaphoreType.DMA,
        ],
    )
    def seg(t_hbm, pk_hbm, out_hbm, pk, sidx, didx, rows, agg, isem, gsem):
        c = lax.axis_index("c")
        s = lax.axis_index("s")
        wid = c * _NS + s

        pltpu.async_copy(pk_hbm.at[wid], pk, isem)

        def _zrow(i, carry):
            for j in range(emb // 16):
                rows[i, pl.ds(j * 16, 16)] = jnp.zeros((16,), jnp.float32)
            return carry

        lax.fori_loop(0, _ZSTG, _zrow, 0)

        def _zcopy(k, carry):
            pltpu.sync_copy(rows.at[pl.ds(0, _ZSTG)],
                            agg.at[pl.ds((s * nz + k) * _ZSTG, _ZSTG)])
            return carry

        lax.fori_loop(0, nz, _zcopy, 0)

        pltpu.make_async_copy(pk_hbm.at[wid], pk, isem).wait()
        plsc.subcore_barrier()

        def _chunk(j, carry):
            for i in range(_CHUNK // 16):
                p = pk[j, pl.ds(i * 16, 16)]
                sidx[pl.ds(i * 16, 16)] = lax.bitwise_and(p, 32767)
                didx[pl.ds(i * 16, 16)] = lax.shift_right_logical(p, 15)
            pltpu.async_copy(t_hbm.at[sidx], rows, gsem).wait()
            pltpu.sync_copy(rows, agg.at[didx], add=True)
            return carry

        lax.fori_loop(0, n_chunks, _chunk, 0)
        plsc.subcore_barrier()

        pltpu.sync_copy(agg.at[pl.ds(s * n_out, n_out)],
                        out_hbm.at[c, pl.ds(s * n_out, n_out)])

    return seg


def _embed_body(blk, x0_ref, x1_ref, kt_ref, vt_ref, m_ref, ee_ref,
                h_ref, t_ref):
    iot = lax.broadcasted_iota(jnp.int32, (blk, 128), 1)
    oh0 = (x0_ref[...] == iot).astype(jnp.float32)
    oh1 = (x1_ref[...] == iot).astype(jnp.float32)
    h = (jnp.dot(oh0, kt_ref[...], preferred_element_type=jnp.float32)
         + jnp.dot(oh1, vt_ref[...], preferred_element_type=jnp.float32))
    h = h * m_ref[...]
    h_ref[...] = h
    t_ref[...] = jnp.maximum(h + ee_ref[...], 0.0)


def _embed_call(x0, x1, kt_pad, vt_pad, mask2d, ee, blk):
    n, emb = mask2d.shape[0], kt_pad.shape[1]
    grid = n // blk
    full = lambda i: (0, 0)
    row = lambda i: (i, 0)
    return pl.pallas_call(
        functools.partial(_embed_body, blk),
        grid=(grid,),
        in_specs=[
            pl.BlockSpec((blk, 1), row),
            pl.BlockSpec((blk, 1), row),
            pl.BlockSpec(kt_pad.shape, full),
            pl.BlockSpec(vt_pad.shape, full),
            pl.BlockSpec((blk, 1), row),
            pl.BlockSpec((1, emb), full),
        ],
        out_specs=[pl.BlockSpec((blk, emb), row),
                   pl.BlockSpec((blk, emb), row)],
        out_shape=[jax.ShapeDtypeStruct((n, emb), jnp.float32),
                   jax.ShapeDtypeStruct((n, emb), jnp.float32)],
    )(x0, x1, kt_pad, vt_pad, mask2d, ee)


def _update_body(is_original, h_ref, p0_ref, p1_ref, m_ref, eps_ref, w1_ref,
                 b1_ref, w2_ref, b2_ref, g_ref, bb_ref, ee_ref,
                 ho_ref, to_ref):
    h = h_ref[...]
    z = h + eps_ref[...] * h + (p0_ref[...] + p1_ref[...])
    a1 = jnp.maximum(
        jnp.dot(z, w1_ref[...], preferred_element_type=jnp.float32)
        + b1_ref[...], 0.0)
    out = (jnp.dot(a1, w2_ref[...], preferred_element_type=jnp.float32)
           + b2_ref[...])
    m = m_ref[...]
    um = m if is_original else 1.0 - m
    pre = um * out + h
    mu = jnp.mean(pre, axis=-1, keepdims=True)
    var = jnp.mean((pre - mu) * (pre - mu), axis=-1, keepdims=True)
    hn = (pre - mu) * lax.rsqrt(var + 1e-5) * g_ref[...] + bb_ref[...]
    ho_ref[...] = hn
    to_ref[...] = jnp.maximum(hn + ee_ref[...], 0.0)


def _update_call(h, p0, p1, mask2d, is_original, eps11, w1, b1, w2, b2,
                 lng, lnb, ee_next, blk):
    n, emb = h.shape
    hid = w1.shape[1]
    grid = n // blk
    full = lambda i: (0, 0)
    row = lambda i: (i, 0)
    return pl.pallas_call(
        functools.partial(_update_body, is_original),
        grid=(grid,),
        in_specs=[
            pl.BlockSpec((blk, emb), row),
            pl.BlockSpec((blk, emb), row),
            pl.BlockSpec((blk, emb), row),
            pl.BlockSpec((blk, 1), row),
            pl.BlockSpec((1, 1), full),
            pl.BlockSpec((emb, hid), full),
            pl.BlockSpec((1, hid), full),
            pl.BlockSpec((hid, emb), full),
            pl.BlockSpec((1, emb), full),
            pl.BlockSpec((1, emb), full),
            pl.BlockSpec((1, emb), full),
            pl.BlockSpec((1, emb), full),
        ],
        out_specs=[pl.BlockSpec((blk, emb), row),
                   pl.BlockSpec((blk, emb), row)],
        out_shape=[jax.ShapeDtypeStruct((n, emb), jnp.float32),
                   jax.ShapeDtypeStruct((n, emb), jnp.float32)],
    )(h, p0, p1, mask2d, eps11, w1, b1, w2, b2, lng, lnb, ee_next)


def _pad_edges(src, dst, n_nodes):
    e = src.shape[0]
    e_pad = _round_up(e, _NW * _CHUNK)
    pad = e_pad - e
    rows_sp = _round_up(n_nodes + 1, _NS * _ZSTG)
    fill = n_nodes + jnp.arange(pad, dtype=jnp.int32) % (rows_sp - n_nodes)
    src_p = jnp.concatenate([src, jnp.zeros((pad,), jnp.int32)])
    dst_p = jnp.concatenate([dst, fill])
    return (dst_p * 32768 + src_p).reshape(_NW, -1, _CHUNK)


def kernel(keys_table, values_table, params, expander_node_mask, x,
           edge_index, expander_edge_index):
    n, emb = x.shape[0], keys_table.shape[1]
    vocab = keys_table.shape[0]
    blk = 2000

    kt_pad = jnp.pad(keys_table, ((0, 128 - vocab), (0, 0)))
    vt_pad = jnp.pad(values_table, ((0, 128 - vocab), (0, 0)))
    mask2d = expander_node_mask[:, None]
    x0 = x[:, 0:1].astype(jnp.int32)
    x1 = x[:, 1:2].astype(jnp.int32)

    e_idx = _pad_edges(edge_index[0], edge_index[1], n)
    l_idx = _pad_edges(expander_edge_index[0], expander_edge_index[1], n)
    r_idx = _pad_edges(expander_edge_index[1], expander_edge_index[0], n)

    convs = []
    for p in params:
        convs.append((p['conv'], e_idx, True, p['ln1_g'], p['ln1_b']))
        convs.append((p['left'], l_idx, False, p['ln2_g'], p['ln2_b']))
        convs.append((p['right'], r_idx, True, p['ln3_g'], p['ln3_b']))

    ee0 = convs[0][0]['edge_emb'][None, :]
    h, t = _embed_call(x0, x1, kt_pad, vt_pad, mask2d, ee0, blk)

    zero_ee = jnp.zeros((1, emb), jnp.float32)
    for k, (cp, eidx, is_orig, lng, lnb) in enumerate(convs):
        seg = _seg_sum_kernel(n, eidx.shape[1], emb)
        partials = seg(t, eidx)[:, :n, :]
        ee_next = (convs[k + 1][0]['edge_emb'][None, :]
                   if k + 1 < len(convs) else zero_ee)
        h, t = _update_call(
            h, partials[0], partials[1], mask2d, is_orig,
            cp['eps'].reshape(1, 1), cp['W1'], cp['b1'][None, :],
            cp['W2'], cp['b2'][None, :], lng[None, :], lnb[None, :],
            ee_next, blk)
    return h


# speedup vs baseline: 1.1310x; 1.1310x over previous
import functools

import jax
import jax.numpy as jnp
from jax import lax
from jax.experimental import pallas as pl
from jax.experimental.pallas import tpu as pltpu
from jax.experimental.pallas import tpu_sc as plsc

_NC = 2
_NS = 16
_NW = _NC * _NS
_CHUNK = 128
_ZSTG = 64


def _round_up(a, b):
    return (a + b - 1) // b * b


@functools.lru_cache(maxsize=None)
def _seg_sum_kernel(n_nodes, n_chunks, emb):
    rows_sp = _round_up(n_nodes + 1, _NS * _ZSTG)
    n_out = rows_sp // _NS
    nz = n_out // _ZSTG

    @functools.partial(
        pl.kernel,
        out_type=jax.ShapeDtypeStruct((_NC, rows_sp, emb), jnp.float32),
        mesh=plsc.VectorSubcoreMesh(core_axis_name="c", subcore_axis_name="s"),
        scratch_types=[
            pltpu.VMEM((n_chunks, _CHUNK), jnp.int32),
            pltpu.VMEM((n_chunks, _CHUNK), jnp.int32),
            pltpu.VMEM((_CHUNK, emb), jnp.float32),
            pltpu.VMEM((_ZSTG, emb), jnp.float32),
            pltpu.VMEM_SHARED((rows_sp, emb), jnp.float32),
            pltpu.SemaphoreType.DMA,
            pltpu.SemaphoreType.DMA,
        ],
    )
    def seg(t_hbm, src_hbm, dst_hbm, out_hbm, srcv, dstv, rows, zbuf, agg,
            gsem, isem):
        c = lax.axis_index("c")
        s = lax.axis_index("s")
        wid = c * _NS + s

        pltpu.async_copy(src_hbm.at[wid], srcv, isem)
        pltpu.async_copy(dst_hbm.at[wid], dstv, isem)

        def _zrow(i, carry):
            for j in range(emb // 16):
                zbuf[i, pl.ds(j * 16, 16)] = jnp.zeros((16,), jnp.float32)
            return carry

        lax.fori_loop(0, _ZSTG, _zrow, 0)

        def _zcopy(k, carry):
            pltpu.sync_copy(zbuf, agg.at[pl.ds((s * nz + k) * _ZSTG, _ZSTG)])
            return carry

        lax.fori_loop(0, nz, _zcopy, 0)

        pltpu.make_async_copy(src_hbm.at[wid], srcv, isem).wait()
        pltpu.make_async_copy(dst_hbm.at[wid], dstv, isem).wait()
        plsc.subcore_barrier()

        def _chunk(j, carry):
            pltpu.async_copy(t_hbm.at[srcv.at[j]], rows, gsem).wait()
            pltpu.sync_copy(rows, agg.at[dstv.at[j]], add=True)
            return carry

        lax.fori_loop(0, n_chunks, _chunk, 0)
        plsc.subcore_barrier()

        pltpu.sync_copy(agg.at[pl.ds(s * n_out, n_out)],
                        out_hbm.at[c, pl.ds(s * n_out, n_out)])

    return seg


def _embed_body(blk, x0_ref, x1_ref, kt_ref, vt_ref, m_ref, ee_ref,
                h_ref, t_ref):
    iot = lax.broadcasted_iota(jnp.int32, (blk, 128), 1)
    oh0 = (x0_ref[...] == iot).astype(jnp.float32)
    oh1 = (x1_ref[...] == iot).astype(jnp.float32)
    h = (jnp.dot(oh0, kt_ref[...], preferred_element_type=jnp.float32)
         + jnp.dot(oh1, vt_ref[...], preferred_element_type=jnp.float32))
    h = h * m_ref[...]
    h_ref[...] = h
    t_ref[...] = jnp.maximum(h + ee_ref[...], 0.0)


def _embed_call(x0, x1, kt_pad, vt_pad, mask2d, ee, blk):
    n, emb = mask2d.shape[0], kt_pad.shape[1]
    grid = n // blk
    full = lambda i: (0, 0)
    row = lambda i: (i, 0)
    return pl.pallas_call(
        functools.partial(_embed_body, blk),
        grid=(grid,),
        in_specs=[
            pl.BlockSpec((blk, 1), row),
            pl.BlockSpec((blk, 1), row),
            pl.BlockSpec(kt_pad.shape, full),
            pl.BlockSpec(vt_pad.shape, full),
            pl.BlockSpec((blk, 1), row),
            pl.BlockSpec((1, emb), full),
        ],
        out_specs=[pl.BlockSpec((blk, emb), row),
                   pl.BlockSpec((blk, emb), row)],
        out_shape=[jax.ShapeDtypeStruct((n, emb), jnp.float32),
                   jax.ShapeDtypeStruct((n, emb), jnp.float32)],
    )(x0, x1, kt_pad, vt_pad, mask2d, ee)


def _update_body(is_original, h_ref, p_ref, m_ref, eps_ref, w1_ref,
                 b1_ref, w2_ref, b2_ref, g_ref, bb_ref, ee_ref,
                 ho_ref, to_ref):
    h = h_ref[...]
    z = h + eps_ref[...] * h + (p_ref[0] + p_ref[1])
    a1 = jnp.maximum(
        jnp.dot(z, w1_ref[...], preferred_element_type=jnp.float32)
        + b1_ref[...], 0.0)
    out = (jnp.dot(a1, w2_ref[...], preferred_element_type=jnp.float32)
           + b2_ref[...])
    m = m_ref[...]
    um = m if is_original else 1.0 - m
    pre = um * out + h
    mu = jnp.mean(pre, axis=-1, keepdims=True)
    var = jnp.mean((pre - mu) * (pre - mu), axis=-1, keepdims=True)
    hn = (pre - mu) * lax.rsqrt(var + 1e-5) * g_ref[...] + bb_ref[...]
    ho_ref[...] = hn
    to_ref[...] = jnp.maximum(hn + ee_ref[...], 0.0)


def _update_call(h, partials, mask2d, is_original, eps11, w1, b1, w2, b2,
                 lng, lnb, ee_next, blk):
    n, emb = h.shape
    hid = w1.shape[1]
    grid = n // blk
    full = lambda i: (0, 0)
    row = lambda i: (i, 0)
    return pl.pallas_call(
        functools.partial(_update_body, is_original),
        grid=(grid,),
        in_specs=[
            pl.BlockSpec((blk, emb), row),
            pl.BlockSpec((2, blk, emb), lambda i: (0, i, 0)),
            pl.BlockSpec((blk, 1), row),
            pl.BlockSpec((1, 1), full),
            pl.BlockSpec((emb, hid), full),
            pl.BlockSpec((1, hid), full),
            pl.BlockSpec((hid, emb), full),
            pl.BlockSpec((1, emb), full),
            pl.BlockSpec((1, emb), full),
            pl.BlockSpec((1, emb), full),
            pl.BlockSpec((1, emb), full),
        ],
        out_specs=[pl.BlockSpec((blk, emb), row),
                   pl.BlockSpec((blk, emb), row)],
        out_shape=[jax.ShapeDtypeStruct((n, emb), jnp.float32),
                   jax.ShapeDtypeStruct((n, emb), jnp.float32)],
    )(h, partials, mask2d, eps11, w1, b1, w2, b2, lng, lnb, ee_next)


def _pad_edges(src, dst, n_nodes):
    e = src.shape[0]
    e_pad = _round_up(e, _NW * _CHUNK)
    pad = e_pad - e
    rows_sp = _round_up(n_nodes + 1, _NS * _ZSTG)
    fill = n_nodes + jnp.arange(pad, dtype=jnp.int32) % (rows_sp - n_nodes)
    src_p = jnp.concatenate([src, jnp.zeros((pad,), jnp.int32)])
    dst_p = jnp.concatenate([dst, fill])
    return (src_p.reshape(_NW, -1, _CHUNK), dst_p.reshape(_NW, -1, _CHUNK))


def kernel(keys_table, values_table, params, expander_node_mask, x,
           edge_index, expander_edge_index):
    n, emb = x.shape[0], keys_table.shape[1]
    vocab = keys_table.shape[0]
    blk = 2000

    kt_pad = jnp.pad(keys_table, ((0, 128 - vocab), (0, 0)))
    vt_pad = jnp.pad(values_table, ((0, 128 - vocab), (0, 0)))
    mask2d = expander_node_mask[:, None]
    x0 = x[:, 0:1].astype(jnp.int32)
    x1 = x[:, 1:2].astype(jnp.int32)

    e_idx = _pad_edges(edge_index[0], edge_index[1], n)
    l_idx = _pad_edges(expander_edge_index[0], expander_edge_index[1], n)
    r_idx = _pad_edges(expander_edge_index[1], expander_edge_index[0], n)

    convs = []
    for p in params:
        convs.append((p['conv'], e_idx, True, p['ln1_g'], p['ln1_b']))
        convs.append((p['left'], l_idx, False, p['ln2_g'], p['ln2_b']))
        convs.append((p['right'], r_idx, True, p['ln3_g'], p['ln3_b']))

    ee0 = convs[0][0]['edge_emb'][None, :]
    h, t = _embed_call(x0, x1, kt_pad, vt_pad, mask2d, ee0, blk)

    zero_ee = jnp.zeros((1, emb), jnp.float32)
    for k, (cp, (src, dst), is_orig, lng, lnb) in enumerate(convs):
        seg = _seg_sum_kernel(n, src.shape[1], emb)
        partials = seg(t, src, dst)
        ee_next = (convs[k + 1][0]['edge_emb'][None, :]
                   if k + 1 < len(convs) else zero_ee)
        h, t = _update_call(
            h, partials, mask2d, is_orig,
            cp['eps'].reshape(1, 1), cp['W1'], cp['b1'][None, :],
            cp['W2'], cp['b2'][None, :], lng[None, :], lnb[None, :],
            ee_next, blk)
    return h


# speedup vs baseline: 1.1329x; 1.0017x over previous
import functools

import jax
import jax.numpy as jnp
from jax import lax
from jax.experimental import pallas as pl
from jax.experimental.pallas import tpu as pltpu
from jax.experimental.pallas import tpu_sc as plsc

_NC = 2
_NS = 16
_NW = _NC * _NS
_CHUNK = 128
_ZSTG = 64


def _round_up(a, b):
    return (a + b - 1) // b * b


@functools.lru_cache(maxsize=None)
def _seg_sum_kernel(n_nodes, n_chunks, emb):
    rows_sp = _round_up(n_nodes + 1, _NS * _ZSTG)
    n_out = rows_sp // _NS
    nz = n_out // _ZSTG

    @functools.partial(
        pl.kernel,
        out_type=jax.ShapeDtypeStruct((_NC, rows_sp, emb), jnp.float32),
        mesh=plsc.VectorSubcoreMesh(core_axis_name="c", subcore_axis_name="s"),
        scratch_types=[
            pltpu.VMEM((n_chunks, _CHUNK), jnp.int32),
            pltpu.VMEM((n_chunks, _CHUNK), jnp.int32),
            pltpu.VMEM((_CHUNK, emb), jnp.float32),
            pltpu.VMEM((_ZSTG, emb), jnp.float32),
            pltpu.VMEM_SHARED((rows_sp, emb), jnp.float32),
            pltpu.SemaphoreType.DMA,
            pltpu.SemaphoreType.DMA,
        ],
    )
    def seg(t_hbm, src_hbm, dst_hbm, out_hbm, srcv, dstv, rows, zbuf, agg,
            gsem, isem):
        c = lax.axis_index("c")
        s = lax.axis_index("s")
        wid = c * _NS + s

        pltpu.async_copy(src_hbm.at[wid], srcv, isem)
        pltpu.async_copy(dst_hbm.at[wid], dstv, isem)

        def _zrow(i, carry):
            for j in range(emb // 16):
                zbuf[i, pl.ds(j * 16, 16)] = jnp.zeros((16,), jnp.float32)
            return carry

        lax.fori_loop(0, _ZSTG, _zrow, 0)

        def _zcopy(k, carry):
            pltpu.sync_copy(zbuf, agg.at[pl.ds((s * nz + k) * _ZSTG, _ZSTG)])
            return carry

        lax.fori_loop(0, nz, _zcopy, 0)

        pltpu.make_async_copy(src_hbm.at[wid], srcv, isem).wait()
        pltpu.make_async_copy(dst_hbm.at[wid], dstv, isem).wait()
        plsc.subcore_barrier()

        def _chunk(j, carry):
            pltpu.async_copy(t_hbm.at[srcv.at[j]], rows, gsem).wait()
            pltpu.sync_copy(rows, agg.at[dstv.at[j]], add=True)
            return carry

        lax.fori_loop(0, n_chunks, _chunk, 0)
        plsc.subcore_barrier()

        pltpu.sync_copy(agg.at[pl.ds(s * n_out, n_out)],
                        out_hbm.at[c, pl.ds(s * n_out, n_out)])

    return seg


def _embed_body(blk, x0_ref, x1_ref, kt_ref, vt_ref, m_ref, ee_ref,
                h_ref, t_ref):
    iot = lax.broadcasted_iota(jnp.int32, (blk, 128), 1)
    oh0 = (x0_ref[...] == iot).astype(jnp.float32)
    oh1 = (x1_ref[...] == iot).astype(jnp.float32)
    h = (jnp.dot(oh0, kt_ref[...], preferred_element_type=jnp.float32)
         + jnp.dot(oh1, vt_ref[...], preferred_element_type=jnp.float32))
    h = h * m_ref[...]
    h_ref[...] = h
    t_ref[...] = jnp.maximum(h + ee_ref[...], 0.0)


def _embed_call(x0, x1, kt_pad, vt_pad, mask2d, ee, blk):
    n, emb = mask2d.shape[0], kt_pad.shape[1]
    grid = n // blk
    full = lambda i: (0, 0)
    row = lambda i: (i, 0)
    return pl.pallas_call(
        functools.partial(_embed_body, blk),
        grid=(grid,),
        in_specs=[
            pl.BlockSpec((blk, 1), row),
            pl.BlockSpec((blk, 1), row),
            pl.BlockSpec(kt_pad.shape, full),
            pl.BlockSpec(vt_pad.shape, full),
            pl.BlockSpec((blk, 1), row),
            pl.BlockSpec((1, emb), full),
        ],
        out_specs=[pl.BlockSpec((blk, emb), row),
                   pl.BlockSpec((blk, emb), row)],
        out_shape=[jax.ShapeDtypeStruct((n, emb), jnp.float32),
                   jax.ShapeDtypeStruct((n, emb), jnp.float32)],
    )(x0, x1, kt_pad, vt_pad, mask2d, ee)


def _update_body(is_original, h_ref, p_ref, m_ref, eps_ref, w1_ref,
                 b1_ref, w2_ref, b2_ref, g_ref, bb_ref, ee_ref,
                 ho_ref, to_ref):
    h = h_ref[...]
    z = h + eps_ref[...] * h + (p_ref[0] + p_ref[1])
    a1 = jnp.maximum(
        jnp.dot(z, w1_ref[...], preferred_element_type=jnp.float32)
        + b1_ref[...], 0.0)
    out = (jnp.dot(a1, w2_ref[...], preferred_element_type=jnp.float32)
           + b2_ref[...])
    m = m_ref[...]
    um = m if is_original else 1.0 - m
    pre = um * out + h
    mu = jnp.mean(pre, axis=-1, keepdims=True)
    var = jnp.mean((pre - mu) * (pre - mu), axis=-1, keepdims=True)
    hn = (pre - mu) * lax.rsqrt(var + 1e-5) * g_ref[...] + bb_ref[...]
    ho_ref[...] = hn
    to_ref[...] = jnp.maximum(hn + ee_ref[...], 0.0)


def _update_call(h, partials, mask2d, is_original, eps11, w1, b1, w2, b2,
                 lng, lnb, ee_next, blk):
    n, emb = h.shape
    hid = w1.shape[1]
    grid = n // blk
    full = lambda i: (0, 0)
    row = lambda i: (i, 0)
    return pl.pallas_call(
        functools.partial(_update_body, is_original),
        grid=(grid,),
        in_specs=[
            pl.BlockSpec((blk, emb), row),
            pl.BlockSpec((2, blk, emb), lambda i: (0, i, 0)),
            pl.BlockSpec((blk, 1), row),
            pl.BlockSpec((1, 1), full),
            pl.BlockSpec((emb, hid), full),
            pl.BlockSpec((1, hid), full),
            pl.BlockSpec((hid, emb), full),
            pl.BlockSpec((1, emb), full),
            pl.BlockSpec((1, emb), full),
            pl.BlockSpec((1, emb), full),
            pl.BlockSpec((1, emb), full),
        ],
        out_specs=[pl.BlockSpec((blk, emb), row),
                   pl.BlockSpec((blk, emb), row)],
        out_shape=[jax.ShapeDtypeStruct((n, emb), jnp.float32),
                   jax.ShapeDtypeStruct((n, emb), jnp.float32)],
    )(h, partials, mask2d, eps11, w1, b1, w2, b2, lng, lnb, ee_next)


def _pad_edges(src, dst, n_nodes):
    e = src.shape[0]
    e_pad = _round_up(e, _NW * _CHUNK)
    pad = e_pad - e
    src_p = jnp.concatenate([src, jnp.zeros((pad,), jnp.int32)])
    dst_p = jnp.concatenate([dst, jnp.full((pad,), n_nodes, jnp.int32)])
    return (src_p.reshape(_NW, -1, _CHUNK), dst_p.reshape(_NW, -1, _CHUNK))


def kernel(keys_table, values_table, params, expander_node_mask, x,
           edge_index, expander_edge_index):
    n, emb = x.shape[0], keys_table.shape[1]
    vocab = keys_table.shape[0]
    blk = 2000

    kt_pad = jnp.pad(keys_table, ((0, 128 - vocab), (0, 0)))
    vt_pad = jnp.pad(values_table, ((0, 128 - vocab), (0, 0)))
    mask2d = expander_node_mask[:, None]
    x0 = x[:, 0:1].astype(jnp.int32)
    x1 = x[:, 1:2].astype(jnp.int32)

    e_idx = _pad_edges(edge_index[0], edge_index[1], n)
    l_idx = _pad_edges(expander_edge_index[0], expander_edge_index[1], n)
    r_idx = _pad_edges(expander_edge_index[1], expander_edge_index[0], n)

    convs = []
    for p in params:
        convs.append((p['conv'], e_idx, True, p['ln1_g'], p['ln1_b']))
        convs.append((p['left'], l_idx, False, p['ln2_g'], p['ln2_b']))
        convs.append((p['right'], r_idx, True, p['ln3_g'], p['ln3_b']))

    ee0 = convs[0][0]['edge_emb'][None, :]
    h, t = _embed_call(x0, x1, kt_pad, vt_pad, mask2d, ee0, blk)

    zero_ee = jnp.zeros((1, emb), jnp.float32)
    for k, (cp, (src, dst), is_orig, lng, lnb) in enumerate(convs):
        seg = _seg_sum_kernel(n, src.shape[1], emb)
        partials = seg(t, src, dst)
        ee_next = (convs[k + 1][0]['edge_emb'][None, :]
                   if k + 1 < len(convs) else zero_ee)
        h, t = _update_call(
            h, partials, mask2d, is_orig,
            cp['eps'].reshape(1, 1), cp['W1'], cp['b1'][None, :],
            cp['W2'], cp['b2'][None, :], lng[None, :], lnb[None, :],
            ee_next, blk)
    return h


# speedup vs baseline: 1.1495x; 1.0146x over previous
import functools

import jax
import jax.numpy as jnp
from jax import lax
from jax.experimental import pallas as pl
from jax.experimental.pallas import tpu as pltpu
from jax.experimental.pallas import tpu_sc as plsc

_NC = 2
_NS = 16
_NW = _NC * _NS
_CHUNK = 128
_ZSTG = 64


def _round_up(a, b):
    return (a + b - 1) // b * b


@functools.lru_cache(maxsize=None)
def _seg_sum_kernel(n_nodes, n_chunks, emb):
    rows_sp = _round_up(n_nodes + 1, _NS * _ZSTG)
    n_out = rows_sp // _NS
    nz = n_out // _ZSTG

    @functools.partial(
        pl.kernel,
        out_type=jax.ShapeDtypeStruct((_NC, rows_sp, emb), jnp.float32),
        mesh=plsc.VectorSubcoreMesh(core_axis_name="c", subcore_axis_name="s"),
        scratch_types=[
            pltpu.VMEM((n_chunks, _CHUNK), jnp.int32),
            pltpu.VMEM((n_chunks, _CHUNK), jnp.int32),
            pltpu.VMEM((_CHUNK, emb), jnp.float32),
            pltpu.VMEM((_ZSTG, emb), jnp.float32),
            pltpu.VMEM_SHARED((rows_sp, emb), jnp.float32),
            pltpu.SemaphoreType.DMA,
            pltpu.SemaphoreType.DMA,
        ],
    )
    def seg(t_hbm, src_hbm, dst_hbm, out_hbm, srcv, dstv, rows, zbuf, agg,
            gsem, isem):
        c = lax.axis_index("c")
        s = lax.axis_index("s")
        wid = c * _NS + s

        pltpu.async_copy(src_hbm.at[wid], srcv, isem)
        pltpu.async_copy(dst_hbm.at[wid], dstv, isem)

        def _zrow(i, carry):
            for j in range(emb // 16):
                zbuf[i, pl.ds(j * 16, 16)] = jnp.zeros((16,), jnp.float32)
            return carry

        lax.fori_loop(0, _ZSTG, _zrow, 0)

        def _zcopy(k, carry):
            pltpu.sync_copy(zbuf, agg.at[pl.ds((s * nz + k) * _ZSTG, _ZSTG)])
            return carry

        lax.fori_loop(0, nz, _zcopy, 0)

        pltpu.make_async_copy(src_hbm.at[wid], srcv, isem).wait()
        pltpu.make_async_copy(dst_hbm.at[wid], dstv, isem).wait()
        plsc.subcore_barrier()

        def _chunk(j, carry):
            pltpu.async_copy(t_hbm.at[srcv.at[j]], rows, gsem).wait()
            pltpu.sync_copy(rows, agg.at[dstv.at[j]], add=True)
            return carry

        lax.fori_loop(0, n_chunks, _chunk, 0)
        plsc.subcore_barrier()

        pltpu.sync_copy(agg.at[pl.ds(s * n_out, n_out)],
                        out_hbm.at[c, pl.ds(s * n_out, n_out)])

    return seg


def _embed_body(blk, x0_ref, x1_ref, kt_ref, vt_ref, m_ref, ee_ref,
                h_ref, t_ref):
    iot = lax.broadcasted_iota(jnp.int32, (blk, 128), 1)
    oh0 = (x0_ref[...] == iot).astype(jnp.float32)
    oh1 = (x1_ref[...] == iot).astype(jnp.float32)
    h = (jnp.dot(oh0, kt_ref[...], preferred_element_type=jnp.float32)
         + jnp.dot(oh1, vt_ref[...], preferred_element_type=jnp.float32))
    h = h * m_ref[...]
    h_ref[...] = h
    t_ref[...] = jnp.maximum(h + ee_ref[...], 0.0)


def _embed_call(x0, x1, kt_pad, vt_pad, mask2d, ee, blk):
    n, emb = mask2d.shape[0], kt_pad.shape[1]
    grid = n // blk
    full = lambda i: (0, 0)
    row = lambda i: (i, 0)
    return pl.pallas_call(
        functools.partial(_embed_body, blk),
        grid=(grid,),
        in_specs=[
            pl.BlockSpec((blk, 1), row),
            pl.BlockSpec((blk, 1), row),
            pl.BlockSpec(kt_pad.shape, full),
            pl.BlockSpec(vt_pad.shape, full),
            pl.BlockSpec((blk, 1), row),
            pl.BlockSpec((1, emb), full),
        ],
        out_specs=[pl.BlockSpec((blk, emb), row),
                   pl.BlockSpec((blk, emb), row)],
        out_shape=[jax.ShapeDtypeStruct((n, emb), jnp.float32),
                   jax.ShapeDtypeStruct((n, emb), jnp.float32)],
    )(x0, x1, kt_pad, vt_pad, mask2d, ee)


def _update_body(is_original, h_ref, p0_ref, p1_ref, m_ref, eps_ref, w1_ref,
                 b1_ref, w2_ref, b2_ref, g_ref, bb_ref, ee_ref,
                 ho_ref, to_ref):
    h = h_ref[...]
    z = h + eps_ref[...] * h + (p0_ref[...] + p1_ref[...])
    a1 = jnp.maximum(
        jnp.dot(z, w1_ref[...], preferred_element_type=jnp.float32)
        + b1_ref[...], 0.0)
    out = (jnp.dot(a1, w2_ref[...], preferred_element_type=jnp.float32)
           + b2_ref[...])
    m = m_ref[...]
    um = m if is_original else 1.0 - m
    pre = um * out + h
    mu = jnp.mean(pre, axis=-1, keepdims=True)
    var = jnp.mean((pre - mu) * (pre - mu), axis=-1, keepdims=True)
    hn = (pre - mu) * lax.rsqrt(var + 1e-5) * g_ref[...] + bb_ref[...]
    ho_ref[...] = hn
    to_ref[...] = jnp.maximum(hn + ee_ref[...], 0.0)


def _update_call(h, p0, p1, mask2d, is_original, eps11, w1, b1, w2, b2,
                 lng, lnb, ee_next, blk):
    n, emb = h.shape
    hid = w1.shape[1]
    grid = n // blk
    full = lambda i: (0, 0)
    row = lambda i: (i, 0)
    return pl.pallas_call(
        functools.partial(_update_body, is_original),
        grid=(grid,),
        in_specs=[
            pl.BlockSpec((blk, emb), row),
            pl.BlockSpec((blk, emb), row),
            pl.BlockSpec((blk, emb), row),
            pl.BlockSpec((blk, 1), row),
            pl.BlockSpec((1, 1), full),
            pl.BlockSpec((emb, hid), full),
            pl.BlockSpec((1, hid), full),
            pl.BlockSpec((hid, emb), full),
            pl.BlockSpec((1, emb), full),
            pl.BlockSpec((1, emb), full),
            pl.BlockSpec((1, emb), full),
            pl.BlockSpec((1, emb), full),
        ],
        out_specs=[pl.BlockSpec((blk, emb), row),
                   pl.BlockSpec((blk, emb), row)],
        out_shape=[jax.ShapeDtypeStruct((n, emb), jnp.float32),
                   jax.ShapeDtypeStruct((n, emb), jnp.float32)],
    )(h, p0, p1, mask2d, eps11, w1, b1, w2, b2, lng, lnb, ee_next)


def _pad_edges(src, dst, n_nodes):
    e = src.shape[0]
    e_pad = _round_up(e, _NW * _CHUNK)
    pad = e_pad - e
    src_p = jnp.concatenate([src, jnp.zeros((pad,), jnp.int32)])
    dst_p = jnp.concatenate([dst, jnp.full((pad,), n_nodes, jnp.int32)])
    return (src_p.reshape(_NW, -1, _CHUNK), dst_p.reshape(_NW, -1, _CHUNK))


def kernel(keys_table, values_table, params, expander_node_mask, x,
           edge_index, expander_edge_index):
    n, emb = x.shape[0], keys_table.shape[1]
    vocab = keys_table.shape[0]
    blk = 2000

    kt_pad = jnp.pad(keys_table, ((0, 128 - vocab), (0, 0)))
    vt_pad = jnp.pad(values_table, ((0, 128 - vocab), (0, 0)))
    mask2d = expander_node_mask[:, None]
    x0 = x[:, 0:1].astype(jnp.int32)
    x1 = x[:, 1:2].astype(jnp.int32)

    e_idx = _pad_edges(edge_index[0], edge_index[1], n)
    l_idx = _pad_edges(expander_edge_index[0], expander_edge_index[1], n)
    r_idx = _pad_edges(expander_edge_index[1], expander_edge_index[0], n)

    convs = []
    for p in params:
        convs.append((p['conv'], e_idx, True, p['ln1_g'], p['ln1_b']))
        convs.append((p['left'], l_idx, False, p['ln2_g'], p['ln2_b']))
        convs.append((p['right'], r_idx, True, p['ln3_g'], p['ln3_b']))

    ee0 = convs[0][0]['edge_emb'][None, :]
    h, t = _embed_call(x0, x1, kt_pad, vt_pad, mask2d, ee0, blk)

    zero_ee = jnp.zeros((1, emb), jnp.float32)
    for k, (cp, (src, dst), is_orig, lng, lnb) in enumerate(convs):
        seg = _seg_sum_kernel(n, src.shape[1], emb)
        partials = seg(t, src, dst)[:, :n, :]
        ee_next = (convs[k + 1][0]['edge_emb'][None, :]
                   if k + 1 < len(convs) else zero_ee)
        h, t = _update_call(
            h, partials[0], partials[1], mask2d, is_orig,
            cp['eps'].reshape(1, 1), cp['W1'], cp['b1'][None, :],
            cp['W2'], cp['b2'][None, :], lng[None, :], lnb[None, :],
            ee_next, blk)
    return h
